# baseline (speedup 1.0000x reference)
import functools

import numpy as np
import jax
import jax.numpy as jnp
from jax import lax
from jax.experimental import pallas as pl
from jax.experimental.pallas import tpu as pltpu
from jax.experimental.pallas import tpu_sc as plsc

VOCAB = 100000
D_MODEL = 1024
MAX_LEN = 8192
B = 4
S = 4096

NC = 2
NS = 16
LANES = 16
NW = NC * NS
PPW = S // NW
CHUNK = 16
NCH = PPW // CHUNK


def _positional_encoding(max_len, d_model):
    pos = np.arange(max_len, dtype=np.float32)[:, None]
    i = np.arange(0, d_model, 2, dtype=np.float32)
    div = np.power(10000.0, i / d_model)
    pe = np.zeros((max_len, d_model), dtype=np.float32)
    pe[:, 0::2] = np.sin(pos / div)
    pe[:, 1::2] = np.cos(pos / div)
    return pe


_PE = _positional_encoding(MAX_LEN, D_MODEL)[:S]


_MESH = plsc.VectorSubcoreMesh(core_axis_name="c", subcore_axis_name="s")


@functools.partial(
    pl.kernel,
    mesh=_MESH,
    out_type=jax.ShapeDtypeStruct((B, S, D_MODEL), jnp.float32),
    scratch_types=(
        [pltpu.VMEM((B, PPW), jnp.int32)]
        + [pltpu.VMEM((CHUNK, D_MODEL), jnp.float32) for _ in range(B)]
        + [pltpu.VMEM((CHUNK * 8, 128), jnp.float32) for _ in range(2)]
        + [pltpu.SemaphoreType.DMA for _ in range(B)]
        + [pltpu.SemaphoreType.DMA for _ in range(B)]
        + [pltpu.SemaphoreType.DMA for _ in range(2)]
    ),
)
def _embed(x_hbm, table_hbm, pe_hbm, out_hbm,
           idx_v, rows0, rows1, rows2, rows3, peb0, peb1,
           g0, g1, g2, g3, o0, o1, o2, o3, ps0, ps1):
    rows_s = (rows0, rows1, rows2, rows3)
    pe_s = (peb0, peb1)
    g_s = (g0, g1, g2, g3)
    o_s = (o0, o1, o2, o3)

    wid = lax.axis_index("s") * NC + lax.axis_index("c")
    p0 = wid * PPW

    for b in range(B):
        pltpu.sync_copy(x_hbm.at[b, pl.ds(p0, PPW)], idx_v.at[b])

    def issue_gather(c, b):
        pltpu.async_copy(
            table_hbm.at[idx_v.at[b, pl.ds(c * CHUNK, CHUNK)]],
            rows_s[b], g_s[b])

    def drain_gather(b):
        pltpu.make_async_copy(
            table_hbm.at[pl.ds(0, CHUNK)], rows_s[b], g_s[b]).wait()

    def issue_pe(c, j):
        pltpu.async_copy(
            pe_hbm.at[pl.ds((p0 + c * CHUNK) * 8, CHUNK * 8)],
            pe_s[j], ps0 if j == 0 else ps1)

    def drain_pe(j):
        pltpu.make_async_copy(
            pe_hbm.at[pl.ds(0, CHUNK * 8)], pe_s[j],
            ps0 if j == 0 else ps1).wait()

    def add_pe(b, j):
        rows_v, pe_v = rows_s[b], pe_s[j]

        def row_step(r, carry):
            r8 = r * 8
            for cc in range(D_MODEL // LANES):
                sl = pl.ds(cc * LANES, LANES)
                plsc.addupdate(rows_v.at[r, sl],
                               pe_v[r8 + cc // 8, pl.ds((cc % 8) * LANES, LANES)])
            return carry

        lax.fori_loop(0, CHUNK, row_step, 0)

    def issue_out(c, b):
        pltpu.async_copy(
            rows_s[b], out_hbm.at[b, pl.ds(p0 + c * CHUNK, CHUNK), :], o_s[b])

    def drain_out(b):
        pltpu.make_async_copy(
            rows_s[b], out_hbm.at[0, pl.ds(0, CHUNK), :], o_s[b]).wait()

    def step(c, b, j, first=False, guard=False):
        drain_gather(b)
        if b == 0:
            drain_pe(j)
        add_pe(b, j)
        if not first:
            drain_out((b + 3) % B)
        cn = c if b == 0 else c + 1
        if guard:
            @pl.when(cn < NCH)
            def _():
                issue_gather(cn, (b + 3) % B)
        else:
            issue_gather(cn, (b + 3) % B)
        issue_out(c, b)

    issue_pe(0, 0)
    issue_pe(1, 1)
    issue_gather(0, 0)
    issue_gather(0, 1)
    issue_gather(0, 2)

    step(0, 0, 0, first=True)
    step(0, 1, 0)
    step(0, 2, 0)
    step(0, 3, 0)
    issue_pe(2, 0)
    step(1, 0, 1)
    step(1, 1, 1)
    step(1, 2, 1)
    step(1, 3, 1)
    issue_pe(3, 1)

    def pair_body(c2, carry):
        c = c2 * 2
        step(c, 0, 0)
        step(c, 1, 0)
        step(c, 2, 0, guard=True)
        step(c, 3, 0, guard=True)

        @pl.when(c + 2 < NCH)
        def _():
            issue_pe(c + 2, 0)

        step(c + 1, 0, 1, guard=True)
        step(c + 1, 1, 1, guard=True)
        step(c + 1, 2, 1, guard=True)
        step(c + 1, 3, 1, guard=True)

        @pl.when(c + 3 < NCH)
        def _():
            issue_pe(c + 3, 1)

        return carry

    lax.fori_loop(1, NCH // 2, pair_body, 0)

    drain_out(3)


def kernel(x, table):
    pe = jnp.asarray(_PE.reshape(S * 8, 128))
    return _embed(x.astype(jnp.int32), table, pe)


# speedup vs baseline: 1.0757x; 1.0757x over previous
import functools

import numpy as np
import jax
import jax.numpy as jnp
from jax import lax
from jax.experimental import pallas as pl
from jax.experimental.pallas import tpu as pltpu
from jax.experimental.pallas import tpu_sc as plsc

VOCAB = 100000
D_MODEL = 1024
MAX_LEN = 8192
B = 4
S = 4096

NC = 2
NS = 16
LANES = 16
NW = NC * NS
PPW = S // NW
CHUNK = 16
NCH = PPW // CHUNK


def _positional_encoding(max_len, d_model):
    pos = np.arange(max_len, dtype=np.float32)[:, None]
    i = np.arange(0, d_model, 2, dtype=np.float32)
    div = np.power(10000.0, i / d_model)
    pe = np.zeros((max_len, d_model), dtype=np.float32)
    pe[:, 0::2] = np.sin(pos / div)
    pe[:, 1::2] = np.cos(pos / div)
    return pe


_PE = _positional_encoding(MAX_LEN, D_MODEL)[:S]


_MESH = plsc.VectorSubcoreMesh(core_axis_name="c", subcore_axis_name="s")


@functools.partial(
    pl.kernel,
    mesh=_MESH,
    out_type=jax.ShapeDtypeStruct((B, S, D_MODEL), jnp.float32),
    scratch_types=(
        [pltpu.VMEM((B, PPW), jnp.int32)]
        + [pltpu.VMEM((CHUNK, D_MODEL), jnp.float32) for _ in range(B)]
        + [pltpu.VMEM((CHUNK, D_MODEL), jnp.float32) for _ in range(2)]
        + [pltpu.SemaphoreType.DMA for _ in range(B)]
        + [pltpu.SemaphoreType.DMA for _ in range(B)]
        + [pltpu.SemaphoreType.DMA for _ in range(2)]
    ),
)
def _embed(x_hbm, table_hbm, pe_hbm, out_hbm,
           idx_v, rows0, rows1, rows2, rows3, peb0, peb1,
           g0, g1, g2, g3, o0, o1, o2, o3, ps0, ps1):
    rows_s = (rows0, rows1, rows2, rows3)
    pe_s = (peb0, peb1)
    g_s = (g0, g1, g2, g3)
    o_s = (o0, o1, o2, o3)

    wid = lax.axis_index("s") * NC + lax.axis_index("c")
    p0 = wid * PPW

    for b in range(B):
        pltpu.sync_copy(x_hbm.at[b, pl.ds(p0, PPW)], idx_v.at[b])

    def issue_gather(c, b):
        pltpu.async_copy(
            table_hbm.at[idx_v.at[b, pl.ds(c * CHUNK, CHUNK)]],
            rows_s[b], g_s[b])

    def drain_gather(b):
        pltpu.make_async_copy(
            table_hbm.at[pl.ds(0, CHUNK)], rows_s[b], g_s[b]).wait()

    def issue_pe(c, j):
        pltpu.async_copy(
            pe_hbm.at[pl.ds(p0 + c * CHUNK, CHUNK)], pe_s[j],
            ps0 if j == 0 else ps1)

    def drain_pe(j):
        pltpu.make_async_copy(
            pe_hbm.at[pl.ds(0, CHUNK)], pe_s[j],
            ps0 if j == 0 else ps1).wait()

    def add_pe(b, j):
        rows_v, pe_v = rows_s[b], pe_s[j]

        def row_step(r, carry):
            for cc in range(D_MODEL // LANES):
                sl = pl.ds(cc * LANES, LANES)
                plsc.addupdate(rows_v.at[r, sl], pe_v[r, sl])
            return carry

        lax.fori_loop(0, CHUNK, row_step, 0)

    def issue_out(c, b):
        pltpu.async_copy(
            rows_s[b], out_hbm.at[b, pl.ds(p0 + c * CHUNK, CHUNK), :], o_s[b])

    def drain_out(b):
        pltpu.make_async_copy(
            rows_s[b], out_hbm.at[0, pl.ds(0, CHUNK), :], o_s[b]).wait()

    def step(c, b, j, first=False, guard=False):
        drain_gather(b)
        if b == 0:
            drain_pe(j)
        add_pe(b, j)
        if not first:
            drain_out((b + 3) % B)
        cn = c if b == 0 else c + 1
        if guard:
            @pl.when(cn < NCH)
            def _():
                issue_gather(cn, (b + 3) % B)
        else:
            issue_gather(cn, (b + 3) % B)
        issue_out(c, b)

    issue_pe(0, 0)
    issue_pe(1, 1)
    issue_gather(0, 0)
    issue_gather(0, 1)
    issue_gather(0, 2)

    step(0, 0, 0, first=True)
    step(0, 1, 0)
    step(0, 2, 0)
    step(0, 3, 0)
    issue_pe(2, 0)
    step(1, 0, 1)
    step(1, 1, 1)
    step(1, 2, 1)
    step(1, 3, 1)
    issue_pe(3, 1)

    def pair_body(c2, carry):
        c = c2 * 2
        step(c, 0, 0)
        step(c, 1, 0)
        step(c, 2, 0, guard=True)
        step(c, 3, 0, guard=True)

        @pl.when(c + 2 < NCH)
        def _():
            issue_pe(c + 2, 0)

        step(c + 1, 0, 1, guard=True)
        step(c + 1, 1, 1, guard=True)
        step(c + 1, 2, 1, guard=True)
        step(c + 1, 3, 1, guard=True)

        @pl.when(c + 3 < NCH)
        def _():
            issue_pe(c + 3, 1)

        return carry

    lax.fori_loop(1, NCH // 2, pair_body, 0)

    drain_out(3)


def kernel(x, table):
    pos = lax.broadcasted_iota(jnp.float32, (S, D_MODEL), 0)
    i = lax.broadcasted_iota(jnp.int32, (S, D_MODEL), 1)
    i_even = (i // 2) * 2
    div = jnp.power(jnp.float32(10000.0), i_even.astype(jnp.float32) / D_MODEL)
    ang = pos / div
    pe = jnp.where(i % 2 == 0, jnp.sin(ang), jnp.cos(ang))
    return _embed(x.astype(jnp.int32), table, pe)


# speedup vs baseline: 1.6700x; 1.5524x over previous
import functools

import numpy as np
import jax
import jax.numpy as jnp
from jax import lax
from jax.experimental import pallas as pl
from jax.experimental.pallas import tpu as pltpu
from jax.experimental.pallas import tpu_sc as plsc

VOCAB = 100000
D_MODEL = 1024
MAX_LEN = 8192
B = 4
S = 4096

NC = 2
NS = 16
LANES = 16
NW = NC * NS
PPW = S // NW
CHUNK = 16
NCH = PPW // CHUNK


def _positional_encoding(max_len, d_model):
    pos = np.arange(max_len, dtype=np.float32)[:, None]
    i = np.arange(0, d_model, 2, dtype=np.float32)
    div = np.power(10000.0, i / d_model)
    pe = np.zeros((max_len, d_model), dtype=np.float32)
    pe[:, 0::2] = np.sin(pos / div)
    pe[:, 1::2] = np.cos(pos / div)
    return pe


_PE = _positional_encoding(MAX_LEN, D_MODEL)[:S]


_MESH = plsc.VectorSubcoreMesh(core_axis_name="c", subcore_axis_name="s")


@functools.partial(
    pl.kernel,
    mesh=_MESH,
    out_type=jax.ShapeDtypeStruct((B, S, D_MODEL), jnp.float32),
    scratch_types=(
        [pltpu.VMEM((B, PPW), jnp.int32)]
        + [pltpu.VMEM((CHUNK, D_MODEL), jnp.float32) for _ in range(B)]
        + [pltpu.VMEM((CHUNK, D_MODEL), jnp.float32) for _ in range(2)]
        + [pltpu.SemaphoreType.DMA for _ in range(B)]
        + [pltpu.SemaphoreType.DMA for _ in range(B)]
        + [pltpu.SemaphoreType.DMA for _ in range(2)]
    ),
)
def _embed(x_hbm, table_hbm, pe_hbm, out_hbm,
           idx_v, rows0, rows1, rows2, rows3, peb0, peb1,
           g0, g1, g2, g3, o0, o1, o2, o3, ps0, ps1):
    rows_s = (rows0, rows1, rows2, rows3)
    pe_s = (peb0, peb1)
    g_s = (g0, g1, g2, g3)
    o_s = (o0, o1, o2, o3)

    wid = lax.axis_index("s") * NC + lax.axis_index("c")
    p0 = wid * PPW

    for b in range(B):
        pltpu.sync_copy(x_hbm.at[b, pl.ds(p0, PPW)], idx_v.at[b])

    def issue_gather(c, b):
        pltpu.async_copy(
            table_hbm.at[idx_v.at[b, pl.ds(c * CHUNK, CHUNK)]],
            rows_s[b], g_s[b])

    def drain_gather(b):
        pltpu.make_async_copy(
            table_hbm.at[pl.ds(0, CHUNK)], rows_s[b], g_s[b]).wait()

    def issue_pe(c, j):
        pltpu.async_copy(
            pe_hbm.at[pl.ds(p0 + c * CHUNK, CHUNK)], pe_s[j],
            ps0 if j == 0 else ps1)

    def drain_pe(j):
        pltpu.make_async_copy(
            pe_hbm.at[pl.ds(0, CHUNK)], pe_s[j],
            ps0 if j == 0 else ps1).wait()

    def add_pe(b, j):
        rows_v, pe_v = rows_s[b], pe_s[j]

        def row_step(r, carry):
            for cc in range(D_MODEL // LANES):
                sl = pl.ds(cc * LANES, LANES)
                plsc.addupdate(rows_v.at[r, sl], pe_v[r, sl])
            return carry

        lax.fori_loop(0, CHUNK, row_step, 0)

    def issue_out(c, b):
        pltpu.async_copy(
            rows_s[b], out_hbm.at[b, pl.ds(p0 + c * CHUNK, CHUNK), :], o_s[b])

    def drain_out(b):
        pltpu.make_async_copy(
            rows_s[b], out_hbm.at[0, pl.ds(0, CHUNK), :], o_s[b]).wait()

    def step(c, b, j, first=False, guard=False):
        drain_gather(b)
        if b == 0:
            drain_pe(j)
        add_pe(b, j)
        if not first:
            drain_out((b + 3) % B)
        cn = c if b == 0 else c + 1
        if guard:
            @pl.when(cn < NCH)
            def _():
                issue_gather(cn, (b + 3) % B)
        else:
            issue_gather(cn, (b + 3) % B)
        issue_out(c, b)

    issue_pe(0, 0)
    issue_pe(1, 1)
    issue_gather(0, 0)
    issue_gather(0, 1)
    issue_gather(0, 2)

    step(0, 0, 0, first=True)
    step(0, 1, 0)
    step(0, 2, 0)
    step(0, 3, 0)
    issue_pe(2, 0)
    step(1, 0, 1)
    step(1, 1, 1)
    step(1, 2, 1)
    step(1, 3, 1)
    issue_pe(3, 1)

    def pair_body(c2, carry):
        c = c2 * 2
        step(c, 0, 0)
        step(c, 1, 0)
        step(c, 2, 0, guard=True)
        step(c, 3, 0, guard=True)

        @pl.when(c + 2 < NCH)
        def _():
            issue_pe(c + 2, 0)

        step(c + 1, 0, 1, guard=True)
        step(c + 1, 1, 1, guard=True)
        step(c + 1, 2, 1, guard=True)
        step(c + 1, 3, 1, guard=True)

        @pl.when(c + 3 < NCH)
        def _():
            issue_pe(c + 3, 1)

        return carry

    lax.fori_loop(1, NCH // 2, pair_body, 0)

    drain_out(3)


def kernel(x, table):
    pe = jnp.asarray(_PE)
    return _embed(x.astype(jnp.int32), table, pe)
